# baseline (speedup 1.0000x reference)
import functools

import jax
import jax.numpy as jnp
from jax import lax
from jax.experimental import pallas as pl
from jax.experimental.pallas import tpu as pltpu
from jax.experimental.pallas import tpu_sc as plsc

EMB = 64
BATCH = 16384
NC = 2
NS = 16
L = 16
NW = NC * NS
BPW = BATCH // NW
CHUNK = 128
NCHUNK = BPW // CHUNK


def _sc_body(qid_hbm, did_hbm, tab_hbm, out_hbm,
             qid_v, did_v, qrows_v, drows_v, out_v, sem):
    wid = lax.axis_index("s") * NC + lax.axis_index("c")
    base = wid * BPW

    pltpu.sync_copy(qid_hbm.at[wid], qid_v)
    pltpu.sync_copy(did_hbm.at[wid], did_v)

    copies = []
    for j in range(NCHUNK):
        dst = qrows_v.at[pl.ds(j * CHUNK, CHUNK)]
        copies.append(pltpu.async_copy(tab_hbm.at[qid_v.at[j]], dst, sem))
        dst = drows_v.at[pl.ds(j * CHUNK, CHUNK)]
        copies.append(pltpu.async_copy(tab_hbm.at[did_v.at[j]], dst, sem))
    for c in copies:
        c.wait()

    def group(g, carry):
        rows = g * L + lax.iota(jnp.int32, L)
        acc = jnp.zeros((L,), jnp.float32)
        for dcol in range(EMB):
            col = jnp.full((L,), dcol, jnp.int32)
            qv = plsc.load_gather(qrows_v, [rows, col])
            dv = plsc.load_gather(drows_v, [rows, col])
            acc = acc + qv * dv
        out_v[pl.ds(g * L, L)] = acc
        return carry

    lax.fori_loop(0, BPW // L, group, 0)

    pltpu.sync_copy(out_v, out_hbm.at[pl.ds(base, BPW)])


@functools.partial(jax.jit, static_argnames=())
def _run(qid, did, id2emb):
    mesh = plsc.VectorSubcoreMesh(core_axis_name="c", subcore_axis_name="s")
    f = functools.partial(
        pl.kernel, mesh=mesh,
        out_type=jax.ShapeDtypeStruct((BATCH,), jnp.float32),
        scratch_types=[
            pltpu.VMEM((NCHUNK, CHUNK), jnp.int32),
            pltpu.VMEM((NCHUNK, CHUNK), jnp.int32),
            pltpu.VMEM((BPW, EMB), jnp.float32),
            pltpu.VMEM((BPW, EMB), jnp.float32),
            pltpu.VMEM((BPW,), jnp.float32),
            pltpu.SemaphoreType.DMA,
        ],
        compiler_params=pltpu.CompilerParams(
            needs_layout_passes=False, use_tc_tiling_on_sc=False),
    )(_sc_body)
    return f(qid, did, id2emb)


def kernel(input_ids, attention_mask, token_type_ids, qid, did,
           session_qid, session_did, session_len, id2emb):
    qid3 = qid.astype(jnp.int32).reshape(NW, NCHUNK, CHUNK)
    did3 = did.astype(jnp.int32).reshape(NW, NCHUNK, CHUNK)
    return _run(qid3, did3, id2emb)


# speedup vs baseline: 2.8641x; 2.8641x over previous
import functools

import jax
import jax.numpy as jnp
from jax import lax
from jax.experimental import pallas as pl
from jax.experimental.pallas import tpu as pltpu
from jax.experimental.pallas import tpu_sc as plsc

VOCAB = 1000000
EMB = 64
BATCH = 16384
NC = 2
NS = 16
L = 16
SLOTS = BATCH // NS
D_PER_CORE = EMB // NC
NCH = SLOTS // 128


def _sc_body(qid_hbm, did_hbm, tabt_hbm, out_hbm,
             qid_v, did_v, row_sp, qv_v, dv_v, acc_v, sem):
    c = lax.axis_index("c")
    s = lax.axis_index("s")

    pltpu.sync_copy(qid_hbm.at[s], qid_v)
    pltpu.sync_copy(did_hbm.at[s], did_v)

    zero = jnp.zeros((L,), jnp.float32)
    for k in range(SLOTS // L):
        acc_v[pl.ds(k * L, L)] = zero

    def d_step(i, carry):
        d = c * D_PER_CORE + i

        @pl.when(s == 0)
        def _():
            pltpu.sync_copy(tabt_hbm.at[d], row_sp)

        plsc.subcore_barrier()

        copies = []
        for j in range(NCH):
            copies.append(pltpu.async_copy(
                row_sp.at[qid_v.at[j]], qv_v.at[pl.ds(j * 128, 128)], sem))
            copies.append(pltpu.async_copy(
                row_sp.at[did_v.at[j]], dv_v.at[pl.ds(j * 128, 128)], sem))
        for cp in copies:
            cp.wait()

        for k in range(SLOTS // L):
            sl = pl.ds(k * L, L)
            acc_v[sl] = acc_v[sl] + qv_v[sl] * dv_v[sl]

        plsc.subcore_barrier()
        return carry

    lax.fori_loop(0, D_PER_CORE, d_step, 0)

    pltpu.sync_copy(acc_v, out_hbm.at[c, pl.ds(s * SLOTS, SLOTS)])


@jax.jit
def _run(qid, did, tabt):
    mesh = plsc.VectorSubcoreMesh(core_axis_name="c", subcore_axis_name="s")
    f = functools.partial(
        pl.kernel, mesh=mesh,
        out_type=jax.ShapeDtypeStruct((NC, BATCH), jnp.float32),
        scratch_types=[
            pltpu.VMEM((NCH, 128), jnp.int32),
            pltpu.VMEM((NCH, 128), jnp.int32),
            pltpu.VMEM_SHARED((VOCAB,), jnp.float32),
            pltpu.VMEM((SLOTS,), jnp.float32),
            pltpu.VMEM((SLOTS,), jnp.float32),
            pltpu.VMEM((SLOTS,), jnp.float32),
            pltpu.SemaphoreType.DMA,
        ],
        compiler_params=pltpu.CompilerParams(
            needs_layout_passes=False, use_tc_tiling_on_sc=True),
    )(_sc_body)
    return f(qid, did, tabt)


def kernel(input_ids, attention_mask, token_type_ids, qid, did,
           session_qid, session_did, session_len, id2emb):
    qid3 = qid.astype(jnp.int32).reshape(NS, NCH, 128)
    did3 = did.astype(jnp.int32).reshape(NS, NCH, 128)
    partial = _run(qid3, did3, id2emb.T)
    return partial[0] + partial[1]


# speedup vs baseline: 3.2084x; 1.1202x over previous
import functools

import jax
import jax.numpy as jnp
from jax import lax
from jax.experimental import pallas as pl
from jax.experimental.pallas import tpu as pltpu
from jax.experimental.pallas import tpu_sc as plsc

VOCAB = 1000000
EMB = 64
BATCH = 16384
NC = 2
NS = 16
L = 16
SLOTS = BATCH // NS
D_PER_CORE = EMB // NC
NPAIR = D_PER_CORE // 2
NCH = SLOTS // 128


def _sc_body(qid_hbm, did_hbm, tabt_hbm, out_hbm,
             qid_v, did_v, buf0, buf1, qv_v, dv_v, acc_v,
             sem0, sem1, semg):
    c = lax.axis_index("c")
    s = lax.axis_index("s")
    d_base = c * D_PER_CORE

    pltpu.sync_copy(qid_hbm.at[s], qid_v)
    pltpu.sync_copy(did_hbm.at[s], did_v)

    zero = jnp.zeros((L,), jnp.float32)
    for k in range(SLOTS // L):
        acc_v[pl.ds(k * L, L)] = zero

    def row_copy(d, buf, sem):
        return pltpu.make_async_copy(tabt_hbm.at[d], buf, sem)

    def process(buf):
        copies = []
        for j in range(NCH):
            copies.append(pltpu.async_copy(
                buf.at[qid_v.at[j]], qv_v.at[pl.ds(j * 128, 128)], semg))
            copies.append(pltpu.async_copy(
                buf.at[did_v.at[j]], dv_v.at[pl.ds(j * 128, 128)], semg))
        for cp in copies:
            cp.wait()
        for k in range(SLOTS // L):
            sl = pl.ds(k * L, L)
            plsc.addupdate(acc_v.at[sl], qv_v[sl] * dv_v[sl])

    @pl.when(s == 0)
    def _():
        row_copy(d_base, buf0, sem0).start()

    def pair(i, carry):
        d0 = d_base + i * 2

        @pl.when(s == 0)
        def _():
            row_copy(d0, buf0, sem0).wait()
            row_copy(d0 + 1, buf1, sem1).start()

        plsc.subcore_barrier()
        process(buf0)
        plsc.subcore_barrier()

        @pl.when(s == 0)
        def _():
            row_copy(d0 + 1, buf1, sem1).wait()

            @pl.when(i + 1 < NPAIR)
            def _():
                row_copy(d0 + 2, buf0, sem0).start()

        plsc.subcore_barrier()
        process(buf1)
        plsc.subcore_barrier()
        return carry

    lax.fori_loop(0, NPAIR, pair, 0)

    pltpu.sync_copy(acc_v, out_hbm.at[c, pl.ds(s * SLOTS, SLOTS)])


@jax.jit
def _run(qid, did, tabt):
    mesh = plsc.VectorSubcoreMesh(core_axis_name="c", subcore_axis_name="s")
    f = functools.partial(
        pl.kernel, mesh=mesh,
        out_type=jax.ShapeDtypeStruct((NC, BATCH), jnp.float32),
        scratch_types=[
            pltpu.VMEM((NCH, 128), jnp.int32),
            pltpu.VMEM((NCH, 128), jnp.int32),
            pltpu.VMEM_SHARED((VOCAB,), jnp.float32),
            pltpu.VMEM_SHARED((VOCAB,), jnp.float32),
            pltpu.VMEM((SLOTS,), jnp.float32),
            pltpu.VMEM((SLOTS,), jnp.float32),
            pltpu.VMEM((SLOTS,), jnp.float32),
            pltpu.SemaphoreType.DMA,
            pltpu.SemaphoreType.DMA,
            pltpu.SemaphoreType.DMA,
        ],
        compiler_params=pltpu.CompilerParams(
            needs_layout_passes=False, use_tc_tiling_on_sc=True),
    )(_sc_body)
    return f(qid, did, tabt)


def kernel(input_ids, attention_mask, token_type_ids, qid, did,
           session_qid, session_did, session_len, id2emb):
    qid3 = qid.astype(jnp.int32).reshape(NS, NCH, 128)
    did3 = did.astype(jnp.int32).reshape(NS, NCH, 128)
    partial = _run(qid3, did3, id2emb.T)
    return partial[0] + partial[1]


# speedup vs baseline: 3.7859x; 1.1800x over previous
import functools

import jax
import jax.numpy as jnp
from jax import lax
from jax.experimental import pallas as pl
from jax.experimental.pallas import tpu as pltpu
from jax.experimental.pallas import tpu_sc as plsc

VOCAB = 1000000
EMB = 64
BATCH = 16384
NC = 2
NS = 16
L = 16
SLOTS = BATCH // NS
D_PER_CORE = EMB // NC
NPAIR = D_PER_CORE // 2
NCH = SLOTS // 128
RCH = (VOCAB // NS) // 128 * 128
TAIL_OFF = RCH * NS
TAIL = 640
VPAD = TAIL_OFF + TAIL


def _sc_body(qid_hbm, did_hbm, tabt_hbm, tail_hbm, out_hbm,
             qid_v, did_v, buf0, buf1, qv_v, dv_v, acc_v,
             sem0, sem1, semg):
    c = lax.axis_index("c")
    s = lax.axis_index("s")
    d_base = c * D_PER_CORE

    pltpu.sync_copy(qid_hbm.at[s], qid_v)
    pltpu.sync_copy(did_hbm.at[s], did_v)

    zero = jnp.zeros((L,), jnp.float32)
    for k in range(SLOTS // L):
        acc_v[pl.ds(k * L, L)] = zero

    def row_copies(d, buf, sem):
        sl = pl.ds(s * RCH, RCH)
        tl = pl.ds(TAIL_OFF, TAIL)
        return (pltpu.make_async_copy(tabt_hbm.at[d].at[sl], buf.at[sl], sem),
                pltpu.make_async_copy(tail_hbm.at[d], buf.at[tl], sem))

    def row_start(d, buf, sem):
        main, tail = row_copies(d, buf, sem)
        main.start()

        @pl.when(s == 0)
        def _():
            tail.start()

    def row_wait(d, buf, sem):
        main, tail = row_copies(d, buf, sem)
        main.wait()

        @pl.when(s == 0)
        def _():
            tail.wait()

    def process(buf):
        copies = []
        for j in range(NCH):
            copies.append(pltpu.async_copy(
                buf.at[qid_v.at[j]], qv_v.at[pl.ds(j * 128, 128)], semg))
            copies.append(pltpu.async_copy(
                buf.at[did_v.at[j]], dv_v.at[pl.ds(j * 128, 128)], semg))
        for cp in copies:
            cp.wait()
        for k in range(SLOTS // L):
            sl = pl.ds(k * L, L)
            plsc.addupdate(acc_v.at[sl], qv_v[sl] * dv_v[sl])

    row_start(d_base, buf0, sem0)

    def pair(i, carry):
        d0 = d_base + i * 2

        row_wait(d0, buf0, sem0)
        row_start(d0 + 1, buf1, sem1)

        plsc.subcore_barrier()
        process(buf0)
        plsc.subcore_barrier()

        row_wait(d0 + 1, buf1, sem1)

        @pl.when(i + 1 < NPAIR)
        def _():
            row_start(d0 + 2, buf0, sem0)

        plsc.subcore_barrier()
        process(buf1)
        plsc.subcore_barrier()
        return carry

    lax.fori_loop(0, NPAIR, pair, 0)

    pltpu.sync_copy(acc_v, out_hbm.at[c, pl.ds(s * SLOTS, SLOTS)])


@jax.jit
def _run(qid, did, tabt, tail):
    mesh = plsc.VectorSubcoreMesh(core_axis_name="c", subcore_axis_name="s")
    f = functools.partial(
        pl.kernel, mesh=mesh,
        out_type=jax.ShapeDtypeStruct((NC, BATCH), jnp.float32),
        scratch_types=[
            pltpu.VMEM((NCH, 128), jnp.int32),
            pltpu.VMEM((NCH, 128), jnp.int32),
            pltpu.VMEM_SHARED((VPAD,), jnp.float32),
            pltpu.VMEM_SHARED((VPAD,), jnp.float32),
            pltpu.VMEM((SLOTS,), jnp.float32),
            pltpu.VMEM((SLOTS,), jnp.float32),
            pltpu.VMEM((SLOTS,), jnp.float32),
            pltpu.SemaphoreType.DMA,
            pltpu.SemaphoreType.DMA,
            pltpu.SemaphoreType.DMA,
        ],
        compiler_params=pltpu.CompilerParams(
            needs_layout_passes=False, use_tc_tiling_on_sc=True),
    )(_sc_body)
    return f(qid, did, tabt, tail)


def kernel(input_ids, attention_mask, token_type_ids, qid, did,
           session_qid, session_did, session_len, id2emb):
    qid3 = qid.astype(jnp.int32).reshape(NS, NCH, 128)
    did3 = did.astype(jnp.int32).reshape(NS, NCH, 128)
    tabt = id2emb.T
    tail = jnp.pad(tabt[:, TAIL_OFF:], ((0, 0), (0, TAIL - (VOCAB - TAIL_OFF))))
    partial = _run(qid3, did3, tabt, tail)
    return partial[0] + partial[1]
